# Initial kernel scaffold; baseline (speedup 1.0000x reference)
import jax
import jax.numpy as jnp
from jax.experimental import pallas as pl


def kernel(idx, targets, table):
    raise NotImplementedError("write your pallas kernel here")


# speedup vs baseline: 1.6656x; 1.6656x over previous
import functools

import jax
import jax.numpy as jnp
from jax import lax
from jax.experimental import pallas as pl
from jax.experimental.pallas import tpu as pltpu
from jax.experimental.pallas import tpu_sc as plsc

VOCAB = 1000
NTOK = 1024 * 50


def _lse_body(table_ref, out_ref):
    x = table_ref[...]
    m = jnp.max(x, axis=1, keepdims=True)
    s = jnp.sum(jnp.exp(x - m), axis=1, keepdims=True)
    out_ref[...] = m + jnp.log(s)


def _lse_tc(table):
    v = table.shape[0]
    return pl.pallas_call(
        _lse_body,
        out_shape=jax.ShapeDtypeStruct((v, 1), jnp.float32),
    )(table)


_NC, _NS, _LANES = 2, 16, 16
_NW = _NC * _NS
_BPW = NTOK // _NW
_CHUNK = 64
_NCHUNK = _BPW // _CHUNK


def _sc_gather_build():
    mesh = plsc.VectorSubcoreMesh(core_axis_name="c", subcore_axis_name="s")

    @functools.partial(
        pl.kernel,
        mesh=mesh,
        compiler_params=pltpu.CompilerParams(
            needs_layout_passes=False, use_tc_tiling_on_sc=False
        ),
        out_type=(
            jax.ShapeDtypeStruct((NTOK, VOCAB), jnp.float32),
            jax.ShapeDtypeStruct((_NW, _LANES), jnp.float32),
        ),
        scratch_types=[
            pltpu.VMEM((_BPW,), jnp.int32),
            pltpu.VMEM((_BPW,), jnp.int32),
            pltpu.VMEM((VOCAB,), jnp.float32),
            pltpu.VMEM((_CHUNK, VOCAB), jnp.float32),
            pltpu.VMEM((_LANES,), jnp.float32),
            pltpu.SemaphoreType.DMA,
        ],
    )
    def k(idx_hbm, tgt_hbm, lse_hbm, table_hbm, out_hbm, part_hbm,
          idx_v, tgt_v, lse_v, rows_v, acc_v, sem):
        wid = lax.axis_index("s") * _NC + lax.axis_index("c")
        base = wid * _BPW
        pltpu.sync_copy(idx_hbm.at[pl.ds(base, _BPW)], idx_v)
        pltpu.sync_copy(tgt_hbm.at[pl.ds(base, _BPW)], tgt_v)
        pltpu.sync_copy(lse_hbm, lse_v)
        acc_v[...] = jnp.zeros((_LANES,), jnp.float32)

        def body(g, carry):
            off = g * _CHUNK
            pltpu.async_copy(
                table_hbm.at[idx_v.at[pl.ds(off, _CHUNK)]], rows_v, sem
            ).wait()
            for j in range(_CHUNK // _LANES):
                rowids = lax.iota(jnp.int32, _LANES) + (j * _LANES)
                tvec = tgt_v[pl.ds(off + j * _LANES, _LANES)]
                ivec = idx_v[pl.ds(off + j * _LANES, _LANES)]
                val = plsc.load_gather(rows_v, [rowids, tvec])
                lse_g = plsc.load_gather(lse_v, [ivec])
                acc_v[...] = acc_v[...] + (lse_g - val)
            pltpu.sync_copy(rows_v, out_hbm.at[pl.ds(base + off, _CHUNK)])
            return carry

        lax.fori_loop(0, _NCHUNK, body, 0)
        pltpu.sync_copy(acc_v, part_hbm.at[wid])

    return k


_sc_gather = _sc_gather_build()


def _loss_body(part_ref, out_ref):
    out_ref[...] = jnp.sum(part_ref[...], keepdims=True).reshape(1, 1) * (
        1.0 / NTOK
    )


def _loss_tc(partials):
    return pl.pallas_call(
        _loss_body,
        out_shape=jax.ShapeDtypeStruct((1, 1), jnp.float32),
    )(partials)


def kernel(idx, targets, table):
    idx_f = idx.reshape(-1).astype(jnp.int32)
    tgt_f = targets.reshape(-1).astype(jnp.int32)
    lse = _lse_tc(table).reshape(VOCAB)
    logits2d, partials = _sc_gather(idx_f, tgt_f, lse, table)
    loss = _loss_tc(partials)[0, 0]
    return (logits2d, loss)


# speedup vs baseline: 1.7039x; 1.0230x over previous
import functools

import jax
import jax.numpy as jnp
from jax import lax
from jax.experimental import pallas as pl
from jax.experimental.pallas import tpu as pltpu
from jax.experimental.pallas import tpu_sc as plsc

VOCAB = 1000
NTOK = 1024 * 50


def _lse_body(table_ref, out_ref):
    x = table_ref[...]
    m = jnp.max(x, axis=1, keepdims=True)
    s = jnp.sum(jnp.exp(x - m), axis=1, keepdims=True)
    out_ref[...] = m + jnp.log(s)


def _lse_tc(table):
    v = table.shape[0]
    return pl.pallas_call(
        _lse_body,
        out_shape=jax.ShapeDtypeStruct((v, 1), jnp.float32),
    )(table)


_NC, _NS, _LANES = 2, 16, 16
_NW = _NC * _NS
_BPW = NTOK // _NW
_CHUNK = 32
_NCHUNK = _BPW // _CHUNK


def _sc_gather_build():
    mesh = plsc.VectorSubcoreMesh(core_axis_name="c", subcore_axis_name="s")

    @functools.partial(
        pl.kernel,
        mesh=mesh,
        compiler_params=pltpu.CompilerParams(
            needs_layout_passes=False, use_tc_tiling_on_sc=False
        ),
        out_type=(
            jax.ShapeDtypeStruct((NTOK, VOCAB), jnp.float32),
            jax.ShapeDtypeStruct((_NW, _LANES), jnp.float32),
        ),
        scratch_types=[
            pltpu.VMEM((_BPW,), jnp.int32),
            pltpu.VMEM((_BPW,), jnp.int32),
            pltpu.VMEM((_CHUNK, VOCAB), jnp.float32),
            pltpu.VMEM((_CHUNK, VOCAB), jnp.float32),
            pltpu.VMEM((VOCAB,), jnp.float32),
            pltpu.VMEM((_LANES,), jnp.float32),
            pltpu.SemaphoreType.DMA,
            pltpu.SemaphoreType.DMA,
            pltpu.SemaphoreType.DMA,
            pltpu.SemaphoreType.DMA,
        ],
    )
    def k(idx_hbm, tgt_hbm, lse_hbm, table_hbm, out_hbm, part_hbm,
          idx_v, tgt_v, rows0, rows1, lse_v, acc_v,
          gsem0, gsem1, wsem0, wsem1):
        rows = (rows0, rows1)
        gsems = (gsem0, gsem1)
        wsems = (wsem0, wsem1)
        wid = lax.axis_index("s") * _NC + lax.axis_index("c")
        base = wid * _BPW
        pltpu.sync_copy(idx_hbm.at[pl.ds(base, _BPW)], idx_v)
        pltpu.sync_copy(tgt_hbm.at[pl.ds(base, _BPW)], tgt_v)
        pltpu.sync_copy(lse_hbm, lse_v)
        acc_v[...] = jnp.zeros((_LANES,), jnp.float32)

        def start_gather(g, p):
            pltpu.make_async_copy(
                table_hbm.at[idx_v.at[pl.ds(g * _CHUNK, _CHUNK)]],
                rows[p], gsems[p],
            ).start()

        def wait_gather(p):
            pltpu.make_async_copy(
                table_hbm.at[idx_v.at[pl.ds(0, _CHUNK)]], rows[p], gsems[p]
            ).wait()

        def start_write(g, p):
            pltpu.make_async_copy(
                rows[p], out_hbm.at[pl.ds(base + g * _CHUNK, _CHUNK)], wsems[p]
            ).start()

        def wait_write(p):
            pltpu.make_async_copy(
                rows[p], out_hbm.at[pl.ds(base, _CHUNK)], wsems[p]
            ).wait()

        def compute(g, p):
            for j in range(_CHUNK // _LANES):
                rowids = lax.iota(jnp.int32, _LANES) + (j * _LANES)
                tvec = tgt_v[pl.ds(g * _CHUNK + j * _LANES, _LANES)]
                ivec = idx_v[pl.ds(g * _CHUNK + j * _LANES, _LANES)]
                val = plsc.load_gather(rows[p], [rowids, tvec])
                lse_g = plsc.load_gather(lse_v, [ivec])
                acc_v[...] = acc_v[...] + (lse_g - val)

        start_gather(0, 0)
        start_gather(1, 1)
        wait_gather(0)
        compute(0, 0)
        start_write(0, 0)
        wait_write(0)
        start_gather(2, 0)
        wait_gather(1)
        compute(1, 1)
        start_write(1, 1)

        def body(i, carry):
            g0 = 2 * i + 2
            wait_write(1)
            start_gather(g0 + 1, 1)
            wait_gather(0)
            compute(g0, 0)
            start_write(g0, 0)
            g1 = g0 + 1
            wait_write(0)
            start_gather(g1 + 1, 0)
            wait_gather(1)
            compute(g1, 1)
            start_write(g1, 1)
            return carry

        lax.fori_loop(0, (_NCHUNK - 4) // 2, body, 0)

        wait_write(1)
        start_gather(_NCHUNK - 1, 1)
        wait_gather(0)
        compute(_NCHUNK - 2, 0)
        start_write(_NCHUNK - 2, 0)
        wait_gather(1)
        compute(_NCHUNK - 1, 1)
        start_write(_NCHUNK - 1, 1)
        wait_write(0)
        wait_write(1)

        pltpu.sync_copy(acc_v, part_hbm.at[wid])

    return k


_sc_gather = _sc_gather_build()


def _loss_body(part_ref, out_ref):
    out_ref[...] = jnp.sum(part_ref[...], keepdims=True).reshape(1, 1) * (
        1.0 / NTOK
    )


def _loss_tc(partials):
    return pl.pallas_call(
        _loss_body,
        out_shape=jax.ShapeDtypeStruct((1, 1), jnp.float32),
    )(partials)


def kernel(idx, targets, table):
    idx_f = idx.reshape(-1).astype(jnp.int32)
    tgt_f = targets.reshape(-1).astype(jnp.int32)
    lse = _lse_tc(table).reshape(VOCAB)
    logits2d, partials = _sc_gather(idx_f, tgt_f, lse, table)
    loss = _loss_tc(partials)[0, 0]
    return (logits2d, loss)


# speedup vs baseline: 2.7978x; 1.6420x over previous
import functools

import jax
import jax.numpy as jnp
from jax import lax
from jax.experimental import pallas as pl
from jax.experimental.pallas import tpu as pltpu
from jax.experimental.pallas import tpu_sc as plsc

VOCAB = 1000
VPAD = 1024
NTOK = 1024 * 50


def _lse_body(table_ref, out_ref):
    x = table_ref[...]
    m = jnp.max(x, axis=1, keepdims=True)
    s = jnp.sum(jnp.exp(x - m), axis=1, keepdims=True)
    out_ref[...] = m + jnp.log(s)


def _lse_tc(table):
    v = table.shape[0]
    return pl.pallas_call(
        _lse_body,
        out_shape=jax.ShapeDtypeStruct((v, 1), jnp.float32),
    )(table)


_NC, _NS, _LANES = 2, 16, 16
_NW = _NC * _NS
_BPW = NTOK // _NW
_CHUNK = 40
_NCHUNK = _BPW // _CHUNK
_VCHUNK = 128
_NVCH = _BPW // _VCHUNK


def _sc_gather_build():
    mesh = plsc.VectorSubcoreMesh(core_axis_name="c", subcore_axis_name="s")

    @functools.partial(
        pl.kernel,
        mesh=mesh,
        compiler_params=pltpu.CompilerParams(
            needs_layout_passes=False, use_tc_tiling_on_sc=True
        ),
        out_type=(
            jax.ShapeDtypeStruct((NTOK, VPAD), jnp.float32),
            jax.ShapeDtypeStruct((_NW, _LANES), jnp.float32),
        ),
        scratch_types=[
            pltpu.VMEM((_BPW,), jnp.int32),
            pltpu.VMEM((_BPW,), jnp.int32),
            pltpu.VMEM((_BPW,), jnp.int32),
            pltpu.VMEM((_BPW,), jnp.float32),
            pltpu.VMEM((_CHUNK, VPAD), jnp.float32),
            pltpu.VMEM((_CHUNK, VPAD), jnp.float32),
            pltpu.VMEM((VOCAB,), jnp.float32),
            pltpu.VMEM((_LANES,), jnp.float32),
            pltpu.SemaphoreType.DMA,
            pltpu.SemaphoreType.DMA,
            pltpu.SemaphoreType.DMA,
            pltpu.SemaphoreType.DMA,
            pltpu.SemaphoreType.DMA,
        ],
    )
    def k(idx_hbm, tgt_hbm, lse_hbm, tpad_hbm, tflat_hbm, out_hbm, part_hbm,
          idx_v, tgt_v, lin_v, vals_v, rows0, rows1, lse_v, acc_v,
          gsem0, gsem1, wsem0, wsem1, vsem):
        rows = (rows0, rows1)
        gsems = (gsem0, gsem1)
        wsems = (wsem0, wsem1)
        wid = lax.axis_index("s") * _NC + lax.axis_index("c")
        base = wid * _BPW
        pltpu.sync_copy(idx_hbm.at[pl.ds(base, _BPW)], idx_v)
        pltpu.sync_copy(tgt_hbm.at[pl.ds(base, _BPW)], tgt_v)
        pltpu.sync_copy(lse_hbm, lse_v)
        acc_v[...] = jnp.zeros((_LANES,), jnp.float32)

        for j in range(_BPW // _LANES):
            sl = pl.ds(j * _LANES, _LANES)
            lin_v[sl] = idx_v[sl] * VOCAB + tgt_v[sl]
        for c in range(_NVCH):
            pltpu.make_async_copy(
                tflat_hbm.at[lin_v.at[pl.ds(c * _VCHUNK, _VCHUNK)]],
                vals_v.at[pl.ds(c * _VCHUNK, _VCHUNK)],
                vsem,
            ).start()
        _TAIL = _BPW - _NVCH * _VCHUNK
        if _TAIL:
            pltpu.make_async_copy(
                tflat_hbm.at[lin_v.at[pl.ds(_NVCH * _VCHUNK, _TAIL)]],
                vals_v.at[pl.ds(_NVCH * _VCHUNK, _TAIL)],
                vsem,
            ).start()

        def start_gather(g, p):
            pltpu.make_async_copy(
                tpad_hbm.at[idx_v.at[pl.ds(g * _CHUNK, _CHUNK)]],
                rows[p], gsems[p],
            ).start()

        def wait_gather(p):
            pltpu.make_async_copy(
                tpad_hbm.at[idx_v.at[pl.ds(0, _CHUNK)]], rows[p], gsems[p]
            ).wait()

        def start_write(g, p):
            pltpu.make_async_copy(
                rows[p], out_hbm.at[pl.ds(base + g * _CHUNK, _CHUNK)], wsems[p]
            ).start()

        def wait_write(p):
            pltpu.make_async_copy(
                rows[p], out_hbm.at[pl.ds(base, _CHUNK)], wsems[p]
            ).wait()

        start_gather(0, 0)
        start_gather(1, 1)
        wait_gather(0)
        start_write(0, 0)
        wait_write(0)
        start_gather(2, 0)
        wait_gather(1)
        start_write(1, 1)

        def body(i, carry):
            g0 = 2 * i + 2
            wait_write(1)
            start_gather(g0 + 1, 1)
            wait_gather(0)
            start_write(g0, 0)
            g1 = g0 + 1
            wait_write(0)
            start_gather(g1 + 1, 0)
            wait_gather(1)
            start_write(g1, 1)
            return carry

        lax.fori_loop(0, (_NCHUNK - 4) // 2, body, 0)

        wait_write(1)
        start_gather(_NCHUNK - 1, 1)
        wait_gather(0)
        start_write(_NCHUNK - 2, 0)
        wait_gather(1)
        start_write(_NCHUNK - 1, 1)

        for c in range(_NVCH):
            pltpu.make_async_copy(
                tflat_hbm.at[lin_v.at[pl.ds(0, _VCHUNK)]],
                vals_v.at[pl.ds(c * _VCHUNK, _VCHUNK)],
                vsem,
            ).wait()
        if _TAIL:
            pltpu.make_async_copy(
                tflat_hbm.at[lin_v.at[pl.ds(0, _TAIL)]],
                vals_v.at[pl.ds(_NVCH * _VCHUNK, _TAIL)],
                vsem,
            ).wait()
        for j in range(_BPW // _LANES):
            sl = pl.ds(j * _LANES, _LANES)
            lse_g = plsc.load_gather(lse_v, [idx_v[sl]])
            acc_v[...] = acc_v[...] + (lse_g - vals_v[sl])

        wait_write(0)
        wait_write(1)
        pltpu.sync_copy(acc_v, part_hbm.at[wid])

    return k


_sc_gather = _sc_gather_build()


def _loss_body(part_ref, out_ref):
    out_ref[...] = jnp.sum(part_ref[...], keepdims=True).reshape(1, 1) * (
        1.0 / NTOK
    )


def _loss_tc(partials):
    return pl.pallas_call(
        _loss_body,
        out_shape=jax.ShapeDtypeStruct((1, 1), jnp.float32),
    )(partials)


def kernel(idx, targets, table):
    idx_f = idx.reshape(-1).astype(jnp.int32)
    tgt_f = targets.reshape(-1).astype(jnp.int32)
    lse = _lse_tc(table).reshape(VOCAB)
    tpad = jnp.pad(table, ((0, 0), (0, VPAD - VOCAB)))
    tflat = table.reshape(-1)
    out_pad, partials = _sc_gather(idx_f, tgt_f, lse, tpad, tflat)
    loss = _loss_tc(partials)[0, 0]
    return (out_pad[:, :VOCAB], loss)
